# baseline (speedup 1.0000x reference)
import functools

import jax
import jax.numpy as jnp
from jax import lax
from jax.experimental import pallas as pl
from jax.experimental.pallas import tpu as pltpu
from jax.experimental.pallas import tpu_sc as plsc


def kernel(x, pe_table):
    seq_len = x.shape[1]
    d = pe_table.shape[1]

    info = plsc.get_sparse_core_info()
    nc, ns = info.num_cores, info.num_subcores
    nw = nc * ns
    rows_per_w = seq_len // nw

    if rows_per_w == 128:
        sizes = [8, 8, 16, 32, 56, 8]
        reuse = {5: 0}
    else:
        sizes = [rows_per_w]
        reuse = {}
    nbufs = len(sizes) - len(reuse)

    mesh = plsc.VectorSubcoreMesh(core_axis_name="c", subcore_axis_name="s")

    @functools.partial(
        pl.kernel,
        mesh=mesh,
        out_type=jax.ShapeDtypeStruct((seq_len, d), jnp.float32),
        scratch_types=(
            [pltpu.VMEM((sizes[i], d), jnp.float32) for i in range(nbufs)]
            + [pltpu.SemaphoreType.DMA] * (2 * nbufs)
        ),
    )
    def copy_rows(table_hbm, out_hbm, *scratch):
        bufs = scratch[:nbufs]
        lsems = scratch[nbufs : 2 * nbufs]
        ssems = scratch[2 * nbufs :]
        wid = lax.axis_index("s") * nc + lax.axis_index("c")
        base = wid * rows_per_w

        offs = [sum(sizes[:i]) for i in range(len(sizes))]
        bidx = [reuse.get(i, min(i, nbufs - 1)) for i in range(len(sizes))]

        def load(c):
            return pltpu.make_async_copy(
                table_hbm.at[pl.ds(base + offs[c], sizes[c])],
                bufs[bidx[c]],
                lsems[bidx[c]],
            )

        def store(c):
            return pltpu.make_async_copy(
                bufs[bidx[c]],
                out_hbm.at[pl.ds(base + offs[c], sizes[c])],
                ssems[bidx[c]],
            )

        for c in range(len(sizes)):
            if c not in reuse:
                load(c).start()
        done_stores = set()
        for c in range(len(sizes)):
            if c in reuse:
                prev = reuse[c]
                store(prev).wait()
                done_stores.add(prev)
                load(c).start()
            load(c).wait()
            store(c).start()
        for c in range(len(sizes)):
            if c not in done_stores:
                store(c).wait()

    return copy_rows(pe_table)[None]


# speedup vs baseline: 1.0038x; 1.0038x over previous
import functools

import jax
import jax.numpy as jnp
from jax import lax
from jax.experimental import pallas as pl
from jax.experimental.pallas import tpu as pltpu
from jax.experimental.pallas import tpu_sc as plsc


def kernel(x, pe_table):
    seq_len = x.shape[1]
    d = pe_table.shape[1]

    info = plsc.get_sparse_core_info()
    nc, ns = info.num_cores, info.num_subcores
    nw = nc * ns
    rows_per_w = seq_len // nw

    if rows_per_w == 128:
        sizes = [8, 16, 32, 64, 8]
        reuse = {4: 0}
    else:
        sizes = [rows_per_w]
        reuse = {}
    nbufs = len(sizes) - len(reuse)

    mesh = plsc.VectorSubcoreMesh(core_axis_name="c", subcore_axis_name="s")

    @functools.partial(
        pl.kernel,
        mesh=mesh,
        out_type=jax.ShapeDtypeStruct((seq_len, d), jnp.float32),
        scratch_types=(
            [pltpu.VMEM((sizes[i], d), jnp.float32) for i in range(nbufs)]
            + [pltpu.SemaphoreType.DMA] * (2 * nbufs)
        ),
    )
    def copy_rows(table_hbm, out_hbm, *scratch):
        bufs = scratch[:nbufs]
        lsems = scratch[nbufs : 2 * nbufs]
        ssems = scratch[2 * nbufs :]
        wid = lax.axis_index("s") * nc + lax.axis_index("c")
        base = wid * rows_per_w

        offs = [sum(sizes[:i]) for i in range(len(sizes))]
        bidx = [reuse.get(i, min(i, nbufs - 1)) for i in range(len(sizes))]

        def load(c):
            return pltpu.make_async_copy(
                table_hbm.at[pl.ds(base + offs[c], sizes[c])],
                bufs[bidx[c]],
                lsems[bidx[c]],
            )

        def store(c):
            return pltpu.make_async_copy(
                bufs[bidx[c]],
                out_hbm.at[pl.ds(base + offs[c], sizes[c])],
                ssems[bidx[c]],
            )

        for c in range(len(sizes)):
            if c not in reuse:
                load(c).start()
        done_stores = set()
        for c in range(len(sizes)):
            if c in reuse:
                prev = reuse[c]
                store(prev).wait()
                done_stores.add(prev)
                load(c).start()
            load(c).wait()
            store(c).start()
        for c in range(len(sizes)):
            if c not in done_stores:
                store(c).wait()

    return copy_rows(pe_table)[None]
